# Initial kernel scaffold; baseline (speedup 1.0000x reference)
import jax
import jax.numpy as jnp
from jax.experimental import pallas as pl


def kernel(edge_index, h, e, snorm_n, snorm_e, emb, W0, W1, b, gamma, beta, mW0, mb0, mW1, mb1, mW2, mb2):
    raise NotImplementedError("write your pallas kernel here")


# speedup vs baseline: 8.0591x; 8.0591x over previous
import functools

import jax
import jax.numpy as jnp
from jax import lax
from jax.experimental import pallas as pl
from jax.experimental.pallas import tpu as pltpu
from jax.experimental.pallas import tpu_sc as plsc

_N = 10000
_E = 320000
_H = 128
_VOCAB = 128
_NC = 6

_NCORES = 2
_NSUB = 16
_NW = _NCORES * _NSUB
_EPW = _E // _NW
_C = 80
_NCHUNK = _EPW // _C
_NP = 10240
_RPS = _NP // _NSUB


def _msg_body(xn_hbm, src_hbm, dst_hbm, zeros_hbm, out_hbm,
              src_v, dst_v, rows_v, agg_sh, gsem):
    c = lax.axis_index("c")
    s = lax.axis_index("s")
    wid = s * _NCORES + c

    pltpu.sync_copy(zeros_hbm.at[pl.ds(s * _RPS, _RPS)],
                    agg_sh.at[pl.ds(s * _RPS, _RPS)])
    pltpu.sync_copy(src_hbm.at[wid], src_v)
    pltpu.sync_copy(dst_hbm.at[wid], dst_v)
    plsc.subcore_barrier()

    def body(j, carry):
        pltpu.async_copy(xn_hbm.at[src_v.at[j]], rows_v, gsem).wait()
        pltpu.sync_copy(rows_v, agg_sh.at[dst_v.at[j]], add=True)
        return carry

    lax.fori_loop(0, _NCHUNK, body, 0, unroll=False)

    plsc.subcore_barrier()
    pltpu.sync_copy(agg_sh.at[pl.ds(s * _RPS, _RPS)],
                    out_hbm.at[c, pl.ds(s * _RPS, _RPS)])


def _deg_body(dst_hbm, ones_hbm, zeros_hbm, out_hbm,
              dst_v, ones_v, deg_sh):
    c = lax.axis_index("c")
    s = lax.axis_index("s")
    wid = s * _NCORES + c

    pltpu.sync_copy(zeros_hbm.at[pl.ds(s * _RPS, _RPS)],
                    deg_sh.at[pl.ds(s * _RPS, _RPS)])
    pltpu.sync_copy(dst_hbm.at[wid], dst_v)
    pltpu.sync_copy(ones_hbm, ones_v)
    plsc.subcore_barrier()

    def body(j, carry):
        pltpu.sync_copy(ones_v, deg_sh.at[dst_v.at[j]], add=True)
        return carry

    lax.fori_loop(0, _NCHUNK, body, 0, unroll=False)

    plsc.subcore_barrier()
    pltpu.sync_copy(deg_sh.at[pl.ds(s * _RPS, _RPS)],
                    out_hbm.at[c, pl.ds(s * _RPS, _RPS)])


@functools.cache
def _sc_kernels():
    mesh = plsc.VectorSubcoreMesh(core_axis_name="c", subcore_axis_name="s",
                                  num_cores=_NCORES, num_subcores=_NSUB)
    msg = pl.kernel(
        _msg_body,
        out_type=jax.ShapeDtypeStruct((_NCORES, _NP, _H), jnp.float32),
        mesh=mesh,
        scratch_types=[
            pltpu.VMEM((_NCHUNK, _C), jnp.int32),
            pltpu.VMEM((_NCHUNK, _C), jnp.int32),
            pltpu.VMEM((_C, _H), jnp.float32),
            pltpu.VMEM_SHARED((_NP, _H), jnp.float32),
            pltpu.SemaphoreType.DMA,
        ],
    )
    deg = pl.kernel(
        _deg_body,
        out_type=jax.ShapeDtypeStruct((_NCORES, _NP, _H), jnp.float32),
        mesh=mesh,
        scratch_types=[
            pltpu.VMEM((_NCHUNK, _C), jnp.int32),
            pltpu.VMEM((_C, _H), jnp.float32),
            pltpu.VMEM_SHARED((_NP, _H), jnp.float32),
        ],
    )
    return msg, deg


def _prep_body(h_ref, emb_ref, degp_ref, x_ref, xn_ref, norm_ref):
    h = h_ref[...]
    oh = (h == lax.broadcasted_iota(jnp.int32, (_N, _VOCAB), 1))
    x = jnp.dot(oh.astype(jnp.float32), emb_ref[...],
                preferred_element_type=jnp.float32)
    deg = degp_ref[0, 0:_N, 0:1] + degp_ref[1, 0:_N, 0:1]
    norm = jnp.where(deg > 0, lax.rsqrt(jnp.maximum(deg, 1.0)), 0.0)
    x_ref[...] = x
    xn_ref[...] = x * norm
    norm_ref[...] = norm


_prep_tc = pl.pallas_call(
    _prep_body,
    out_shape=[
        jax.ShapeDtypeStruct((_N, _H), jnp.float32),
        jax.ShapeDtypeStruct((_N, _H), jnp.float32),
        jax.ShapeDtypeStruct((_N, 1), jnp.float32),
    ],
)


def _dense_core(x, p_ref, norm, W0_ref, W1_ref, b_ref, g_ref, bt_ref, sn_ref):
    agg = (p_ref[0, 0:_N, :] + p_ref[1, 0:_N, :]) * norm
    t = (jnp.dot(x, W0_ref[...], preferred_element_type=jnp.float32)
         - jnp.dot(agg, W1_ref[...], preferred_element_type=jnp.float32)
         + b_ref[...])
    t = t * sn_ref[...]
    mu = jnp.mean(t, axis=0, keepdims=True)
    tc = t - mu
    var = jnp.mean(tc * tc, axis=0, keepdims=True)
    th = tc * lax.rsqrt(var + 1e-5) * g_ref[...] + bt_ref[...]
    return x + jnp.maximum(th, 0.0)


def _dense_body(x_ref, p_ref, norm_ref, W0_ref, W1_ref, b_ref, g_ref, bt_ref,
                sn_ref, xo_ref, xno_ref):
    norm = norm_ref[...]
    xo = _dense_core(x_ref[...], p_ref, norm, W0_ref, W1_ref, b_ref, g_ref,
                     bt_ref, sn_ref)
    xo_ref[...] = xo
    xno_ref[...] = xo * norm


_dense_tc = pl.pallas_call(
    _dense_body,
    out_shape=[
        jax.ShapeDtypeStruct((_N, _H), jnp.float32),
        jax.ShapeDtypeStruct((_N, _H), jnp.float32),
    ],
)


def _final_body(x_ref, p_ref, norm_ref, W0_ref, W1_ref, b_ref, g_ref, bt_ref,
                sn_ref, mW0_ref, mb0_ref, mW1_ref, mb1_ref, mW2_ref, mb2_ref,
                out_ref):
    xo = _dense_core(x_ref[...], p_ref, norm_ref[...], W0_ref, W1_ref, b_ref,
                     g_ref, bt_ref, sn_ref)
    y = jnp.maximum(jnp.dot(xo, mW0_ref[...],
                            preferred_element_type=jnp.float32)
                    + mb0_ref[...], 0.0)
    y = jnp.maximum(jnp.dot(y, mW1_ref[...],
                            preferred_element_type=jnp.float32)
                    + mb1_ref[...], 0.0)
    out_ref[...] = (jnp.dot(y, mW2_ref[...],
                            preferred_element_type=jnp.float32)
                    + mb2_ref[...])


_final_tc = pl.pallas_call(
    _final_body,
    out_shape=jax.ShapeDtypeStruct((_N, _NC), jnp.float32),
)


def kernel(edge_index, h, e, snorm_n, snorm_e, emb, W0, W1, b, gamma, beta,
           mW0, mb0, mW1, mb1, mW2, mb2):
    src3 = edge_index[0].astype(jnp.int32).reshape(_NW, _NCHUNK, _C)
    dst3 = edge_index[1].astype(jnp.int32).reshape(_NW, _NCHUNK, _C)
    zeros_h = jnp.zeros((_NP, _H), jnp.float32)
    ones_r = jnp.ones((_C, _H), jnp.float32)
    h2d = h.astype(jnp.int32).reshape(_N, 1)
    _msg_kernel, _deg_kernel = _sc_kernels()

    degp = _deg_kernel(dst3, ones_r, zeros_h)
    x, xn, norm = _prep_tc(h2d, emb, degp)

    for l in range(3):
        p = _msg_kernel(xn, src3, dst3, zeros_h)
        x, xn = _dense_tc(x, p, norm, W0[l], W1[l], b[l].reshape(1, _H),
                          gamma[l].reshape(1, _H), beta[l].reshape(1, _H),
                          snorm_n)

    p = _msg_kernel(xn, src3, dst3, zeros_h)
    out = _final_tc(x, p, norm, W0[3], W1[3], b[3].reshape(1, _H),
                    gamma[3].reshape(1, _H), beta[3].reshape(1, _H),
                    snorm_n, mW0, mb0.reshape(1, _H // 2),
                    mW1, mb1.reshape(1, _H // 4), mW2, mb2.reshape(1, _NC))
    return out
